# Initial kernel scaffold; baseline (speedup 1.0000x reference)
import jax
import jax.numpy as jnp
from jax.experimental import pallas as pl


def kernel(data, user_factors, item_factors):
    raise NotImplementedError("write your pallas kernel here")


# speedup vs baseline: 4.5065x; 4.5065x over previous
import functools

import jax
import jax.numpy as jnp
from jax import lax
from jax.experimental import pallas as pl
from jax.experimental.pallas import tpu as pltpu
from jax.experimental.pallas import tpu_sc as plsc

_B = 16384
_D = 3
_NC, _NS, _L = 2, 16, 16
_NW = _NC * _NS
_BPW = _B // _NW
_STEPS = _BPW // _L

_U_ROWS = 1500
_I_ROWS = 2000

_mesh = plsc.VectorSubcoreMesh(core_axis_name="c", subcore_axis_name="s")


@functools.partial(
    pl.kernel,
    out_type=jax.ShapeDtypeStruct((_B,), jnp.float32),
    mesh=_mesh,
    compiler_params=pltpu.CompilerParams(needs_layout_passes=False),
    scratch_types=[
        pltpu.VMEM((_BPW,), jnp.int32),
        pltpu.VMEM((_BPW,), jnp.int32),
        pltpu.VMEM((_U_ROWS * _D,), jnp.float32),
        pltpu.VMEM((_I_ROWS * _D,), jnp.float32),
        pltpu.VMEM((_BPW,), jnp.float32),
    ],
)
def _mf_kernel(uidx_hbm, iidx_hbm, ut_hbm, it_hbm, out_hbm,
               uidx_v, iidx_v, ut_v, it_v, out_v):
    wid = lax.axis_index("s") * _NC + lax.axis_index("c")
    base = wid * _BPW

    pltpu.sync_copy(uidx_hbm.at[pl.ds(base, _BPW)], uidx_v)
    pltpu.sync_copy(iidx_hbm.at[pl.ds(base, _BPW)], iidx_v)
    pltpu.sync_copy(ut_hbm, ut_v)
    pltpu.sync_copy(it_hbm, it_v)

    for step in range(_STEPS):
        off = step * _L
        iu = uidx_v[pl.ds(off, _L)] * _D
        iv = iidx_v[pl.ds(off, _L)] * _D
        acc = plsc.load_gather(ut_v, [iu]) * plsc.load_gather(it_v, [iv])
        for d in range(1, _D):
            acc = acc + (plsc.load_gather(ut_v, [iu + d]) *
                         plsc.load_gather(it_v, [iv + d]))
        out_v[pl.ds(off, _L)] = acc

    pltpu.sync_copy(out_v, out_hbm.at[pl.ds(base, _BPW)])


def kernel(data, user_factors, item_factors):
    uidx = data[0].astype(jnp.int32)
    iidx = data[1].astype(jnp.int32)
    ut = user_factors.reshape(-1)
    it = item_factors.reshape(-1)
    return _mf_kernel(uidx, iidx, ut, it)


# speedup vs baseline: 4.5505x; 1.0098x over previous
import functools

import jax
import jax.numpy as jnp
from jax import lax
from jax.experimental import pallas as pl
from jax.experimental.pallas import tpu as pltpu
from jax.experimental.pallas import tpu_sc as plsc

_B = 16384
_D = 3
_NC, _NS, _L = 2, 16, 16
_NW = _NC * _NS
_BPW = _B // _NW
_STEPS = _BPW // _L

_U_ROWS = 1500
_I_ROWS = 2000

_mesh = plsc.VectorSubcoreMesh(core_axis_name="c", subcore_axis_name="s")


@functools.partial(
    pl.kernel,
    out_type=jax.ShapeDtypeStruct((_B,), jnp.float32),
    mesh=_mesh,
    compiler_params=pltpu.CompilerParams(needs_layout_passes=False,
                                         use_tc_tiling_on_sc=False),
    scratch_types=[
        pltpu.VMEM((_BPW,), jnp.int32),
        pltpu.VMEM((_BPW,), jnp.int32),
        pltpu.VMEM((_U_ROWS, _D), jnp.float32),
        pltpu.VMEM((_I_ROWS, _D), jnp.float32),
        pltpu.VMEM((_BPW,), jnp.float32),
        pltpu.SemaphoreType.DMA,
    ],
)
def _mf_kernel(data_hbm, ut_hbm, it_hbm, out_hbm,
               uidx_v, iidx_v, ut_v, it_v, out_v, sem):
    wid = lax.axis_index("s") * _NC + lax.axis_index("c")
    base = wid * _BPW

    cps = [
        pltpu.async_copy(data_hbm.at[0, pl.ds(base, _BPW)], uidx_v, sem),
        pltpu.async_copy(data_hbm.at[1, pl.ds(base, _BPW)], iidx_v, sem),
        pltpu.async_copy(ut_hbm, ut_v, sem),
        pltpu.async_copy(it_hbm, it_v, sem),
    ]
    for cp in cps:
        cp.wait()

    for step in range(_STEPS):
        off = step * _L
        iu = uidx_v[pl.ds(off, _L)]
        iv = iidx_v[pl.ds(off, _L)]
        acc = None
        for d in range(_D):
            dcol = jnp.full((_L,), d, jnp.int32)
            prod = (plsc.load_gather(ut_v, [iu, dcol]) *
                    plsc.load_gather(it_v, [iv, dcol]))
            acc = prod if acc is None else acc + prod
        out_v[pl.ds(off, _L)] = acc

    pltpu.sync_copy(out_v, out_hbm.at[pl.ds(base, _BPW)])


def kernel(data, user_factors, item_factors):
    return _mf_kernel(data.astype(jnp.int32), user_factors, item_factors)


# speedup vs baseline: 5.2644x; 1.1569x over previous
import functools

import jax
import jax.numpy as jnp
from jax import lax
from jax.experimental import pallas as pl
from jax.experimental.pallas import tpu as pltpu
from jax.experimental.pallas import tpu_sc as plsc

_B = 16384
_D = 3
_NC, _NS, _L = 2, 16, 16
_NW = _NC * _NS
_BPW = _B // _NW
_STEPS = _BPW // _L

_U_ROWS = 1500
_I_ROWS = 2000
_IT_BASE = 4504
_CHUNK = 664
_TAB = _CHUNK * _NS

_mesh = plsc.VectorSubcoreMesh(core_axis_name="c", subcore_axis_name="s")


@functools.partial(
    pl.kernel,
    out_type=jax.ShapeDtypeStruct((_B,), jnp.float32),
    mesh=_mesh,
    compiler_params=pltpu.CompilerParams(needs_layout_passes=False,
                                         use_tc_tiling_on_sc=False),
    scratch_types=[
        pltpu.VMEM((2, _BPW), jnp.int32),
        pltpu.VMEM_SHARED((_TAB,), jnp.float32),
        pltpu.VMEM((_TAB,), jnp.float32),
        pltpu.VMEM((_BPW,), jnp.float32),
        pltpu.SemaphoreType.DMA,
    ],
)
def _mf_kernel(data_hbm, tabs_hbm, out_hbm,
               idx_v, tabs_sh, tabs_v, out_v, sem):
    s = lax.axis_index("s")
    wid = s * _NC + lax.axis_index("c")
    base = wid * _BPW

    idx_cp = pltpu.async_copy(data_hbm.at[:, pl.ds(base, _BPW)], idx_v, sem)

    chunk = s * _CHUNK
    pltpu.sync_copy(tabs_hbm.at[pl.ds(chunk, _CHUNK)],
                    tabs_sh.at[pl.ds(chunk, _CHUNK)])
    plsc.subcore_barrier()
    pltpu.sync_copy(tabs_sh, tabs_v)
    idx_cp.wait()

    for step in range(_STEPS):
        off = step * _L
        iu = idx_v[0, pl.ds(off, _L)] * _D
        iv = idx_v[1, pl.ds(off, _L)] * _D + _IT_BASE
        acc = plsc.load_gather(tabs_v, [iu]) * plsc.load_gather(tabs_v, [iv])
        for d in range(1, _D):
            acc = acc + (plsc.load_gather(tabs_v, [iu + d]) *
                         plsc.load_gather(tabs_v, [iv + d]))
        out_v[pl.ds(off, _L)] = acc

    pltpu.sync_copy(out_v, out_hbm.at[pl.ds(base, _BPW)])


def kernel(data, user_factors, item_factors):
    uf = user_factors.reshape(-1)
    itf = item_factors.reshape(-1)
    tabs = jnp.concatenate([
        uf,
        jnp.zeros((_IT_BASE - _U_ROWS * _D,), jnp.float32),
        itf,
        jnp.zeros((_TAB - _IT_BASE - _I_ROWS * _D,), jnp.float32),
    ])
    return _mf_kernel(data.astype(jnp.int32), tabs)
